# baseline (speedup 1.0000x reference)
import functools

import jax
import jax.numpy as jnp
from jax.experimental import pallas as pl
from jax.experimental.pallas import tpu as pltpu

_N = 10000
_B = 4


def _combine_body(r_ref, agg_ref, extra_ref, o_ref):
    o_ref[...] = jax.nn.relu(r_ref[...] + agg_ref[...]) + extra_ref[...]


def _combine(r, agg, extra):
    M, F = r.shape
    blk = 2000
    return pl.pallas_call(
        _combine_body,
        grid=(M // blk,),
        in_specs=[
            pl.BlockSpec((blk, F), lambda i: (i, 0)),
            pl.BlockSpec((blk, F), lambda i: (i, 0)),
            pl.BlockSpec((blk, F), lambda i: (i, 0)),
        ],
        out_specs=pl.BlockSpec((blk, F), lambda i: (i, 0)),
        out_shape=jax.ShapeDtypeStruct((M, F), r.dtype),
    )(r, agg, extra)


def kernel(points, cs, edge, W1r, W1n, b1, W2r, W2n, b2, W3r, W3n, b3, W4r, W4n, b4):
    B, N, _ = points.shape
    X = points.reshape(B * N, 4)
    C = cs.reshape(B * N, cs.shape[-1])
    src = edge[:, 0]
    dst = edge[:, 1]
    offs = (jnp.arange(B, dtype=jnp.int32) * N)[:, None]
    src_full = (src[None, :] + offs).reshape(-1)
    dst_full = (dst[None, :] + offs).reshape(-1)

    def layer(x, Wr, Wn, b):
        y = x @ Wn
        r = x @ Wr + b
        agg = jnp.zeros_like(r).at[dst_full].add(y[src_full])
        return r, agg

    r1, a1 = layer(X, W1r, W1n, b1)
    x1 = _combine(r1, a1, C)
    r2, a2 = layer(x1, W2r, W2n, b2)
    x2 = _combine(r2, a2, x1)
    r3, a3 = layer(x2, W3r, W3n, b3)
    x3 = _combine(r3, a3, x2)
    r4, a4 = layer(x3, W4r, W4n, b4)
    x4 = jnp.tanh(r4 + a4)
    return x4.reshape(B, N, 4)


# speedup vs baseline: 5.4692x; 5.4692x over previous
import functools

import jax
import jax.numpy as jnp
from jax import lax
from jax.experimental import pallas as pl
from jax.experimental.pallas import tpu as pltpu
from jax.experimental.pallas import tpu_sc as plsc

_N = 10000
_E = 160000
_H = 256
_B = 4
_NC = 2
_NS = 16
_NR = 4
_RW = 2560
_CH = 80
_CAP = 126
_NP = 10240
_STR = _RW // _NS
_BLK = 2048
_H4 = 32
_TRASH = _RW

_f32 = jnp.float32


def _tc1_body(pts_ref, wr_ref, wn_ref, b_ref, y_ref, r_ref):
    x = pts_ref[0]
    y = jnp.dot(x, wn_ref[...], preferred_element_type=_f32)
    r = jnp.dot(x, wr_ref[...], preferred_element_type=_f32) + b_ref[...]
    y_ref[0, 0] = y[:, :128]
    y_ref[0, 1] = y[:, 128:]
    r_ref[0, 0] = r[:, :128]
    r_ref[0, 1] = r[:, 128:]


def _hspec():
    return pl.BlockSpec((1, _NC, _BLK, 128), lambda bi, i: (bi, 0, i, 0))


def _hshape():
    return jax.ShapeDtypeStruct((_B, _NC, _NP, 128), _f32)


def _tc1(points, Wr, Wn, b):
    return pl.pallas_call(
        _tc1_body,
        grid=(_B, _NP // _BLK),
        in_specs=[
            pl.BlockSpec((1, _BLK, 4), lambda bi, i: (bi, i, 0)),
            pl.BlockSpec((4, _H), lambda bi, i: (0, 0)),
            pl.BlockSpec((4, _H), lambda bi, i: (0, 0)),
            pl.BlockSpec((1, _H), lambda bi, i: (0, 0)),
        ],
        out_specs=[_hspec(), _hspec()],
        out_shape=[_hshape(), _hshape()],
    )(points, Wr, Wn, b)


def _tc_mid_body(gc_ref, ex_ref, wr_ref, wn_ref, b_ref, x_ref, y_ref, r_ref):
    x0 = jax.nn.relu(gc_ref[0, 0]) + ex_ref[0, 0]
    x1 = jax.nn.relu(gc_ref[0, 1]) + ex_ref[0, 1]
    x = jnp.concatenate([x0, x1], axis=1)
    y = jnp.dot(x, wn_ref[...], preferred_element_type=_f32)
    r = jnp.dot(x, wr_ref[...], preferred_element_type=_f32) + b_ref[...]
    x_ref[0, 0] = x0
    x_ref[0, 1] = x1
    y_ref[0, 0] = y[:, :128]
    y_ref[0, 1] = y[:, 128:]
    r_ref[0, 0] = r[:, :128]
    r_ref[0, 1] = r[:, 128:]


def _tc_mid(gc, extra, Wr, Wn, b):
    return pl.pallas_call(
        _tc_mid_body,
        grid=(_B, _NP // _BLK),
        in_specs=[
            _hspec(),
            _hspec(),
            pl.BlockSpec((_H, _H), lambda bi, i: (0, 0)),
            pl.BlockSpec((_H, _H), lambda bi, i: (0, 0)),
            pl.BlockSpec((1, _H), lambda bi, i: (0, 0)),
        ],
        out_specs=[_hspec(), _hspec(), _hspec()],
        out_shape=[_hshape(), _hshape(), _hshape()],
    )(gc, extra, Wr, Wn, b)


def _tc4_body(gc_ref, ex_ref, wr_ref, wn_ref, b_ref, y_ref, r_ref):
    x0 = jax.nn.relu(gc_ref[0, 0]) + ex_ref[0, 0]
    x1 = jax.nn.relu(gc_ref[0, 1]) + ex_ref[0, 1]
    x = jnp.concatenate([x0, x1], axis=1)
    y_ref[0] = jnp.dot(x, wn_ref[...], preferred_element_type=_f32)
    r_ref[0] = jnp.dot(x, wr_ref[...], preferred_element_type=_f32) + b_ref[...]


def _tc4(gc, extra, Wr, Wn, b):
    return pl.pallas_call(
        _tc4_body,
        grid=(_B, _NP // _BLK),
        in_specs=[
            _hspec(),
            _hspec(),
            pl.BlockSpec((_H, _H4), lambda bi, i: (0, 0)),
            pl.BlockSpec((_H, _H4), lambda bi, i: (0, 0)),
            pl.BlockSpec((1, _H4), lambda bi, i: (0, 0)),
        ],
        out_specs=[
            pl.BlockSpec((1, _BLK, _H4), lambda bi, i: (bi, i, 0)),
            pl.BlockSpec((1, _BLK, _H4), lambda bi, i: (bi, i, 0)),
        ],
        out_shape=[
            jax.ShapeDtypeStruct((_B, _NP, _H4), _f32),
            jax.ShapeDtypeStruct((_B, _NP, _H4), _f32),
        ],
    )(gc, extra, Wr, Wn, b)


def _tc5_body(gc_ref, o_ref):
    o_ref[0] = jnp.tanh(gc_ref[0, :, :4])


def _tc5(gc):
    return pl.pallas_call(
        _tc5_body,
        grid=(_B, _N // 2000),
        in_specs=[pl.BlockSpec((1, 2000, _H4), lambda bi, i: (bi, i, 0))],
        out_specs=pl.BlockSpec((1, 2000, 4), lambda bi, i: (bi, i, 0)),
        out_shape=jax.ShapeDtypeStruct((_B, _N, 4), _f32),
    )(gc)


_mesh = plsc.VectorSubcoreMesh(
    core_axis_name="c", subcore_axis_name="s", num_cores=_NC, num_subcores=_NS
)

_SC_SCRATCH = [
    pltpu.VMEM((_CAP, _CH), jnp.int32),
    pltpu.VMEM((_CAP, _CH), jnp.int32),
    pltpu.VMEM((16,), jnp.int32),
    pltpu.VMEM((_CH, 128), _f32),
    pltpu.VMEM((_CH, 128), _f32),
    pltpu.VMEM_SHARED((_RW + 8, 128), _f32),
    pltpu.SemaphoreType.DMA,
    pltpu.SemaphoreType.DMA,
]


def _range_pass(p, ysl, rsl, osl, srcm, dstm, ncnt, src_v, dst_v, cnt_v,
                rows0, rows1, agg, sem0, sem1, s):
    base = pl.multiple_of(p * _RW + s * _STR, 8)
    pltpu.sync_copy(srcm.at[p].at[s], src_v)
    pltpu.sync_copy(dstm.at[p].at[s], dst_v)
    pltpu.sync_copy(ncnt.at[p].at[s], cnt_v)
    pltpu.sync_copy(rsl.at[pl.ds(base, _STR)], agg.at[pl.ds(s * _STR, _STR)])
    n = jnp.max(cnt_v[...])
    plsc.subcore_barrier()
    pltpu.async_copy(ysl.at[src_v.at[0]], rows0, sem0)
    pltpu.async_copy(ysl.at[src_v.at[1]], rows1, sem1)

    def chunk_pair(i, carry):
        j0 = 2 * i
        pltpu.make_async_copy(ysl.at[src_v.at[j0]], rows0, sem0).wait()
        pltpu.sync_copy(rows0, agg.at[dst_v.at[j0]], add=True)
        pltpu.async_copy(ysl.at[src_v.at[lax.rem(j0 + 2, n)]], rows0, sem0)
        j1 = j0 + 1
        pltpu.make_async_copy(ysl.at[src_v.at[j1]], rows1, sem1).wait()
        pltpu.sync_copy(rows1, agg.at[dst_v.at[j1]], add=True)
        pltpu.async_copy(ysl.at[src_v.at[lax.rem(j1 + 2, n)]], rows1, sem1)
        return carry

    lax.fori_loop(0, n // 2, chunk_pair, 0)
    pltpu.make_async_copy(ysl.at[src_v.at[0]], rows0, sem0).wait()
    pltpu.make_async_copy(ysl.at[src_v.at[1]], rows1, sem1).wait()
    plsc.subcore_barrier()
    pltpu.sync_copy(agg.at[pl.ds(s * _STR, _STR)], osl.at[pl.ds(base, _STR)])
    plsc.subcore_barrier()


@functools.partial(
    pl.kernel,
    out_type=jax.ShapeDtypeStruct((_B, _NC, _NP, 128), _f32),
    mesh=_mesh,
    scratch_types=_SC_SCRATCH,
    compiler_params=pltpu.CompilerParams(needs_layout_passes=False),
)
def _sc_mid(y_hbm, r_hbm, srcm, dstm, ncnt, out_hbm, src_v, dst_v, cnt_v,
            rows0, rows1, agg, sem0, sem1):
    c = lax.axis_index("c")
    s = lax.axis_index("s")
    for b in range(_B):
        for u in range(4):
            h = u // 2
            if h == 0:
                p = c + 2 * (u % 2)
            else:
                p = (1 - c) + 2 * (u % 2)
            _range_pass(p, y_hbm.at[b].at[h], r_hbm.at[b].at[h],
                        out_hbm.at[b].at[h], srcm, dstm, ncnt, src_v, dst_v,
                        cnt_v, rows0, rows1, agg, sem0, sem1, s)


@functools.partial(
    pl.kernel,
    out_type=jax.ShapeDtypeStruct((_NP, 128), _f32),
    mesh=_mesh,
    scratch_types=_SC_SCRATCH,
    compiler_params=pltpu.CompilerParams(needs_layout_passes=False),
)
def _sc4(y_hbm, r_hbm, srcm, dstm, ncnt, out_hbm, src_v, dst_v, cnt_v,
         rows0, rows1, agg, sem0, sem1):
    c = lax.axis_index("c")
    s = lax.axis_index("s")
    for k in range(_NR // _NC):
        p = c + _NC * k
        _range_pass(p, y_hbm, r_hbm, out_hbm, srcm, dstm, ncnt, src_v, dst_v,
                    cnt_v, rows0, rows1, agg, sem0, sem1, s)


def _prep_edges(edge):
    src = edge[:, 0]
    dst = edge[:, 1]
    q = dst // _RW
    dstl = dst - q * _RW
    onehot = q[None, :] == jnp.arange(_NR, dtype=jnp.int32)[:, None]
    ranks = jnp.cumsum(onehot.astype(jnp.int32), axis=1) - 1
    rank = ranks[q, jnp.arange(_E)]
    g = rank // _CH
    tile = g % _NS
    slot = g // _NS
    flat = ((q * _NS + tile) * _CAP + slot) * _CH + rank % _CH
    size = _NR * _NS * _CAP * _CH
    pad_src = (jnp.arange(size, dtype=jnp.int32) % 640)
    srcm = pad_src.at[flat].set(src).reshape(_NR, _NS, _CAP, _CH)
    dstm = (jnp.full((size,), _TRASH, jnp.int32).at[flat].set(dstl)
            .reshape(_NR, _NS, _CAP, _CH))
    counts = jnp.sum(onehot, axis=1)
    nch = (counts + _CH - 1) // _CH
    svec = jnp.arange(_NS, dtype=jnp.int32)
    cnt = jnp.clip((nch[:, None] - svec[None, :] + _NS - 1) // _NS, 0)
    cnt = jnp.maximum(cnt + cnt % 2, 2)
    ncnt = jnp.broadcast_to(cnt[:, :, None], (_NR, _NS, 16)).astype(jnp.int32)
    return srcm, dstm, ncnt


def kernel(points, cs, edge, W1r, W1n, b1, W2r, W2n, b2, W3r, W3n, b3,
           W4r, W4n, b4):
    srcm, dstm, ncnt = _prep_edges(edge)
    pts_p = jnp.pad(points, ((0, 0), (0, _NP - _N), (0, 0)))
    css = jnp.pad(cs.reshape(_B, _N, _NC, 128).transpose(0, 2, 1, 3),
                  ((0, 0), (0, 0), (0, _NP - _N), (0, 0)))

    y1, r1 = _tc1(pts_p, W1r, W1n, b1.reshape(1, -1))
    gc1 = _sc_mid(y1, r1, srcm, dstm, ncnt)
    x1, y2, r2 = _tc_mid(gc1, css, W2r, W2n, b2.reshape(1, -1))
    gc2 = _sc_mid(y2, r2, srcm, dstm, ncnt)
    x2, y3, r3 = _tc_mid(gc2, x1, W3r, W3n, b3.reshape(1, -1))
    gc3 = _sc_mid(y3, r3, srcm, dstm, ncnt)

    W4r_p = jnp.zeros((_H, _H4), _f32).at[:, :4].set(W4r)
    W4n_p = jnp.zeros((_H, _H4), _f32).at[:, :4].set(W4n)
    b4_p = jnp.zeros((1, _H4), _f32).at[0, :4].set(b4)
    y4, r4 = _tc4(gc3, x2, W4r_p, W4n_p, b4_p)
    y4p = y4.transpose(1, 0, 2).reshape(_NP, _B * _H4)
    r4p = r4.transpose(1, 0, 2).reshape(_NP, _B * _H4)
    gc4p = _sc4(y4p, r4p, srcm, dstm, ncnt)
    gc4 = gc4p.reshape(_NP, _B, _H4).transpose(1, 0, 2)
    return _tc5(gc4)


# speedup vs baseline: 5.7552x; 1.0523x over previous
import functools

import jax
import jax.numpy as jnp
from jax import lax
from jax.experimental import pallas as pl
from jax.experimental.pallas import tpu as pltpu
from jax.experimental.pallas import tpu_sc as plsc

_N = 10000
_E = 160000
_H = 256
_B = 4
_NC = 2
_NS = 16
_NR = 4
_RW = 2560
_CH = 128
_CAP = 80
_NP = 10240
_STR = _RW // _NS
_BLK = 2048
_H4 = 32
_TRASH = _RW

_f32 = jnp.float32


def _tc1_body(pts_ref, wr_ref, wn_ref, b_ref, y_ref, r_ref):
    x = pts_ref[0]
    y = jnp.dot(x, wn_ref[...], preferred_element_type=_f32)
    r = jnp.dot(x, wr_ref[...], preferred_element_type=_f32) + b_ref[...]
    y_ref[0, 0] = y[:, :128]
    y_ref[0, 1] = y[:, 128:]
    r_ref[0, 0] = r[:, :128]
    r_ref[0, 1] = r[:, 128:]


def _hspec():
    return pl.BlockSpec((1, _NC, _BLK, 128), lambda bi, i: (bi, 0, i, 0))


def _hshape():
    return jax.ShapeDtypeStruct((_B, _NC, _NP, 128), _f32)


def _tc1(points, Wr, Wn, b):
    return pl.pallas_call(
        _tc1_body,
        grid=(_B, _NP // _BLK),
        in_specs=[
            pl.BlockSpec((1, _BLK, 4), lambda bi, i: (bi, i, 0)),
            pl.BlockSpec((4, _H), lambda bi, i: (0, 0)),
            pl.BlockSpec((4, _H), lambda bi, i: (0, 0)),
            pl.BlockSpec((1, _H), lambda bi, i: (0, 0)),
        ],
        out_specs=[_hspec(), _hspec()],
        out_shape=[_hshape(), _hshape()],
    )(points, Wr, Wn, b)


def _tc_mid_body(gc_ref, ex_ref, wr_ref, wn_ref, b_ref, x_ref, y_ref, r_ref):
    x0 = jax.nn.relu(gc_ref[0, 0]) + ex_ref[0, 0]
    x1 = jax.nn.relu(gc_ref[0, 1]) + ex_ref[0, 1]
    x = jnp.concatenate([x0, x1], axis=1)
    y = jnp.dot(x, wn_ref[...], preferred_element_type=_f32)
    r = jnp.dot(x, wr_ref[...], preferred_element_type=_f32) + b_ref[...]
    x_ref[0, 0] = x0
    x_ref[0, 1] = x1
    y_ref[0, 0] = y[:, :128]
    y_ref[0, 1] = y[:, 128:]
    r_ref[0, 0] = r[:, :128]
    r_ref[0, 1] = r[:, 128:]


def _tc_mid(gc, extra, Wr, Wn, b):
    return pl.pallas_call(
        _tc_mid_body,
        grid=(_B, _NP // _BLK),
        in_specs=[
            _hspec(),
            _hspec(),
            pl.BlockSpec((_H, _H), lambda bi, i: (0, 0)),
            pl.BlockSpec((_H, _H), lambda bi, i: (0, 0)),
            pl.BlockSpec((1, _H), lambda bi, i: (0, 0)),
        ],
        out_specs=[_hspec(), _hspec(), _hspec()],
        out_shape=[_hshape(), _hshape(), _hshape()],
    )(gc, extra, Wr, Wn, b)


def _tc4_body(gc_ref, ex_ref, wr_ref, wn_ref, b_ref, y_ref, r_ref):
    x0 = jax.nn.relu(gc_ref[0, 0]) + ex_ref[0, 0]
    x1 = jax.nn.relu(gc_ref[0, 1]) + ex_ref[0, 1]
    x = jnp.concatenate([x0, x1], axis=1)
    y_ref[0] = jnp.dot(x, wn_ref[...], preferred_element_type=_f32)
    r_ref[0] = jnp.dot(x, wr_ref[...], preferred_element_type=_f32) + b_ref[...]


def _tc4(gc, extra, Wr, Wn, b):
    return pl.pallas_call(
        _tc4_body,
        grid=(_B, _NP // _BLK),
        in_specs=[
            _hspec(),
            _hspec(),
            pl.BlockSpec((_H, _H4), lambda bi, i: (0, 0)),
            pl.BlockSpec((_H, _H4), lambda bi, i: (0, 0)),
            pl.BlockSpec((1, _H4), lambda bi, i: (0, 0)),
        ],
        out_specs=[
            pl.BlockSpec((1, _BLK, _H4), lambda bi, i: (bi, i, 0)),
            pl.BlockSpec((1, _BLK, _H4), lambda bi, i: (bi, i, 0)),
        ],
        out_shape=[
            jax.ShapeDtypeStruct((_B, _NP, _H4), _f32),
            jax.ShapeDtypeStruct((_B, _NP, _H4), _f32),
        ],
    )(gc, extra, Wr, Wn, b)


def _tc5_body(gc_ref, o_ref):
    o_ref[0] = jnp.tanh(gc_ref[0, :, :4])


def _tc5(gc):
    return pl.pallas_call(
        _tc5_body,
        grid=(_B, _N // 2000),
        in_specs=[pl.BlockSpec((1, 2000, _H4), lambda bi, i: (bi, i, 0))],
        out_specs=pl.BlockSpec((1, 2000, 4), lambda bi, i: (bi, i, 0)),
        out_shape=jax.ShapeDtypeStruct((_B, _N, 4), _f32),
    )(gc)


_mesh = plsc.VectorSubcoreMesh(
    core_axis_name="c", subcore_axis_name="s", num_cores=_NC, num_subcores=_NS
)

_SC_SCRATCH = [
    pltpu.VMEM((_CAP, _CH), jnp.int32),
    pltpu.VMEM((_CAP, _CH), jnp.int32),
    pltpu.VMEM((16,), jnp.int32),
    pltpu.VMEM((_CH, 128), _f32),
    pltpu.VMEM((_CH, 128), _f32),
    pltpu.VMEM_SHARED((_RW + 8, 128), _f32),
    pltpu.SemaphoreType.DMA,
    pltpu.SemaphoreType.DMA,
]


def _range_pass(p, ysl, rsl, osl, srcm, dstm, ncnt, src_v, dst_v, cnt_v,
                rows0, rows1, agg, sem0, sem1, s):
    base = pl.multiple_of(p * _RW + s * _STR, 8)
    pltpu.sync_copy(srcm.at[p].at[s], src_v)
    pltpu.sync_copy(dstm.at[p].at[s], dst_v)
    pltpu.sync_copy(ncnt.at[p].at[s], cnt_v)
    pltpu.sync_copy(rsl.at[pl.ds(base, _STR)], agg.at[pl.ds(s * _STR, _STR)])
    n = jnp.max(cnt_v[...])
    plsc.subcore_barrier()
    pltpu.async_copy(ysl.at[src_v.at[0]], rows0, sem0)
    pltpu.async_copy(ysl.at[src_v.at[1]], rows1, sem1)

    def chunk_pair(i, carry):
        j0 = 2 * i
        pltpu.make_async_copy(ysl.at[src_v.at[j0]], rows0, sem0).wait()
        pltpu.sync_copy(rows0, agg.at[dst_v.at[j0]], add=True)
        pltpu.async_copy(ysl.at[src_v.at[lax.rem(j0 + 2, n)]], rows0, sem0)
        j1 = j0 + 1
        pltpu.make_async_copy(ysl.at[src_v.at[j1]], rows1, sem1).wait()
        pltpu.sync_copy(rows1, agg.at[dst_v.at[j1]], add=True)
        pltpu.async_copy(ysl.at[src_v.at[lax.rem(j1 + 2, n)]], rows1, sem1)
        return carry

    lax.fori_loop(0, n // 2, chunk_pair, 0)
    pltpu.make_async_copy(ysl.at[src_v.at[0]], rows0, sem0).wait()
    pltpu.make_async_copy(ysl.at[src_v.at[1]], rows1, sem1).wait()
    plsc.subcore_barrier()
    pltpu.sync_copy(agg.at[pl.ds(s * _STR, _STR)], osl.at[pl.ds(base, _STR)])
    plsc.subcore_barrier()


@functools.partial(
    pl.kernel,
    out_type=jax.ShapeDtypeStruct((_B, _NC, _NP, 128), _f32),
    mesh=_mesh,
    scratch_types=_SC_SCRATCH,
    compiler_params=pltpu.CompilerParams(needs_layout_passes=False),
)
def _sc_mid(y_hbm, r_hbm, srcm, dstm, ncnt, out_hbm, src_v, dst_v, cnt_v,
            rows0, rows1, agg, sem0, sem1):
    c = lax.axis_index("c")
    s = lax.axis_index("s")
    for b in range(_B):
        for u in range(4):
            h = u // 2
            if h == 0:
                p = c + 2 * (u % 2)
            else:
                p = (1 - c) + 2 * (u % 2)
            _range_pass(p, y_hbm.at[b].at[h], r_hbm.at[b].at[h],
                        out_hbm.at[b].at[h], srcm, dstm, ncnt, src_v, dst_v,
                        cnt_v, rows0, rows1, agg, sem0, sem1, s)


@functools.partial(
    pl.kernel,
    out_type=jax.ShapeDtypeStruct((_NP, 128), _f32),
    mesh=_mesh,
    scratch_types=_SC_SCRATCH,
    compiler_params=pltpu.CompilerParams(needs_layout_passes=False),
)
def _sc4(y_hbm, r_hbm, srcm, dstm, ncnt, out_hbm, src_v, dst_v, cnt_v,
         rows0, rows1, agg, sem0, sem1):
    c = lax.axis_index("c")
    s = lax.axis_index("s")
    for k in range(_NR // _NC):
        p = c + _NC * k
        _range_pass(p, y_hbm, r_hbm, out_hbm, srcm, dstm, ncnt, src_v, dst_v,
                    cnt_v, rows0, rows1, agg, sem0, sem1, s)


def _prep_edges(edge):
    src = edge[:, 0]
    dst = edge[:, 1]
    q = dst // _RW
    dstl = dst - q * _RW
    onehot = q[None, :] == jnp.arange(_NR, dtype=jnp.int32)[:, None]
    ranks = jnp.cumsum(onehot.astype(jnp.int32), axis=1) - 1
    rank = ranks[q, jnp.arange(_E)]
    g = rank // _CH
    tile = g % _NS
    slot = g // _NS
    flat = ((q * _NS + tile) * _CAP + slot) * _CH + rank % _CH
    size = _NR * _NS * _CAP * _CH
    pad_src = (jnp.arange(size, dtype=jnp.int32) % 640)
    srcm = pad_src.at[flat].set(src).reshape(_NR, _NS, _CAP, _CH)
    dstm = (jnp.full((size,), _TRASH, jnp.int32).at[flat].set(dstl)
            .reshape(_NR, _NS, _CAP, _CH))
    counts = jnp.sum(onehot, axis=1)
    nch = (counts + _CH - 1) // _CH
    svec = jnp.arange(_NS, dtype=jnp.int32)
    cnt = jnp.clip((nch[:, None] - svec[None, :] + _NS - 1) // _NS, 0)
    cnt = jnp.maximum(cnt + cnt % 2, 2)
    ncnt = jnp.broadcast_to(cnt[:, :, None], (_NR, _NS, 16)).astype(jnp.int32)
    return srcm, dstm, ncnt


def kernel(points, cs, edge, W1r, W1n, b1, W2r, W2n, b2, W3r, W3n, b3,
           W4r, W4n, b4):
    srcm, dstm, ncnt = _prep_edges(edge)
    pts_p = jnp.pad(points, ((0, 0), (0, _NP - _N), (0, 0)))
    css = jnp.pad(cs.reshape(_B, _N, _NC, 128).transpose(0, 2, 1, 3),
                  ((0, 0), (0, 0), (0, _NP - _N), (0, 0)))

    y1, r1 = _tc1(pts_p, W1r, W1n, b1.reshape(1, -1))
    gc1 = _sc_mid(y1, r1, srcm, dstm, ncnt)
    x1, y2, r2 = _tc_mid(gc1, css, W2r, W2n, b2.reshape(1, -1))
    gc2 = _sc_mid(y2, r2, srcm, dstm, ncnt)
    x2, y3, r3 = _tc_mid(gc2, x1, W3r, W3n, b3.reshape(1, -1))
    gc3 = _sc_mid(y3, r3, srcm, dstm, ncnt)

    W4r_p = jnp.zeros((_H, _H4), _f32).at[:, :4].set(W4r)
    W4n_p = jnp.zeros((_H, _H4), _f32).at[:, :4].set(W4n)
    b4_p = jnp.zeros((1, _H4), _f32).at[0, :4].set(b4)
    y4, r4 = _tc4(gc3, x2, W4r_p, W4n_p, b4_p)
    y4p = y4.transpose(1, 0, 2).reshape(_NP, _B * _H4)
    r4p = r4.transpose(1, 0, 2).reshape(_NP, _B * _H4)
    gc4p = _sc4(y4p, r4p, srcm, dstm, ncnt)
    gc4 = gc4p.reshape(_NP, _B, _H4).transpose(1, 0, 2)
    return _tc5(gc4)


# speedup vs baseline: 5.7580x; 1.0005x over previous
import functools

import jax
import jax.numpy as jnp
from jax import lax
from jax.experimental import pallas as pl
from jax.experimental.pallas import tpu as pltpu
from jax.experimental.pallas import tpu_sc as plsc

_N = 10000
_E = 160000
_H = 256
_B = 4
_NC = 2
_NS = 16
_NR = 4
_RW = 2560
_CH = 128
_CAP = 80
_NP = 10240
_STR = _RW // _NS
_BLK = 2048
_H4 = 32
_TRASH = _RW

_f32 = jnp.float32


def _tc1_body(pts_ref, wr_ref, wn_ref, b_ref, y_ref, r_ref):
    x = pts_ref[0]
    y = jnp.dot(x, wn_ref[...], preferred_element_type=_f32)
    r = jnp.dot(x, wr_ref[...], preferred_element_type=_f32) + b_ref[...]
    y_ref[0, 0] = y[:, :128]
    y_ref[0, 1] = y[:, 128:]
    r_ref[0, 0] = r[:, :128]
    r_ref[0, 1] = r[:, 128:]


def _hspec():
    return pl.BlockSpec((1, _NC, _BLK, 128), lambda bi, i: (bi, 0, i, 0))


def _hshape():
    return jax.ShapeDtypeStruct((_B, _NC, _NP, 128), _f32)


def _tc1(points, Wr, Wn, b):
    return pl.pallas_call(
        _tc1_body,
        grid=(_B, _NP // _BLK),
        in_specs=[
            pl.BlockSpec((1, _BLK, 4), lambda bi, i: (bi, i, 0)),
            pl.BlockSpec((4, _H), lambda bi, i: (0, 0)),
            pl.BlockSpec((4, _H), lambda bi, i: (0, 0)),
            pl.BlockSpec((1, _H), lambda bi, i: (0, 0)),
        ],
        out_specs=[_hspec(), _hspec()],
        out_shape=[_hshape(), _hshape()],
    )(points, Wr, Wn, b)


def _tc_mid_body(gc_ref, ex_ref, wr_ref, wn_ref, b_ref, x_ref, y_ref, r_ref):
    x0 = jax.nn.relu(gc_ref[0, 0]) + ex_ref[0, 0]
    x1 = jax.nn.relu(gc_ref[0, 1]) + ex_ref[0, 1]
    x = jnp.concatenate([x0, x1], axis=1)
    y = jnp.dot(x, wn_ref[...], preferred_element_type=_f32)
    r = jnp.dot(x, wr_ref[...], preferred_element_type=_f32) + b_ref[...]
    x_ref[0, 0] = x0
    x_ref[0, 1] = x1
    y_ref[0, 0] = y[:, :128]
    y_ref[0, 1] = y[:, 128:]
    r_ref[0, 0] = r[:, :128]
    r_ref[0, 1] = r[:, 128:]


def _tc_mid(gc, extra, Wr, Wn, b):
    return pl.pallas_call(
        _tc_mid_body,
        grid=(_B, _NP // _BLK),
        in_specs=[
            _hspec(),
            _hspec(),
            pl.BlockSpec((_H, _H), lambda bi, i: (0, 0)),
            pl.BlockSpec((_H, _H), lambda bi, i: (0, 0)),
            pl.BlockSpec((1, _H), lambda bi, i: (0, 0)),
        ],
        out_specs=[_hspec(), _hspec(), _hspec()],
        out_shape=[_hshape(), _hshape(), _hshape()],
    )(gc, extra, Wr, Wn, b)


def _tc4_body(gc_ref, ex_ref, wr_ref, wn_ref, b_ref, y_ref, r_ref):
    x0 = jax.nn.relu(gc_ref[0, 0]) + ex_ref[0, 0]
    x1 = jax.nn.relu(gc_ref[0, 1]) + ex_ref[0, 1]
    x = jnp.concatenate([x0, x1], axis=1)
    y_ref[0] = jnp.dot(x, wn_ref[...], preferred_element_type=_f32)
    r_ref[0] = jnp.dot(x, wr_ref[...], preferred_element_type=_f32) + b_ref[...]


def _tc4(gc, extra, Wr, Wn, b):
    return pl.pallas_call(
        _tc4_body,
        grid=(_B, _NP // _BLK),
        in_specs=[
            _hspec(),
            _hspec(),
            pl.BlockSpec((_H, _H4), lambda bi, i: (0, 0)),
            pl.BlockSpec((_H, _H4), lambda bi, i: (0, 0)),
            pl.BlockSpec((1, _H4), lambda bi, i: (0, 0)),
        ],
        out_specs=[
            pl.BlockSpec((1, _BLK, _H4), lambda bi, i: (bi, i, 0)),
            pl.BlockSpec((1, _BLK, _H4), lambda bi, i: (bi, i, 0)),
        ],
        out_shape=[
            jax.ShapeDtypeStruct((_B, _NP, _H4), _f32),
            jax.ShapeDtypeStruct((_B, _NP, _H4), _f32),
        ],
    )(gc, extra, Wr, Wn, b)


def _tc5_body(gc_ref, o_ref):
    o_ref[0] = jnp.tanh(gc_ref[0, :, :4])


def _tc5(gc):
    return pl.pallas_call(
        _tc5_body,
        grid=(_B, _N // 2000),
        in_specs=[pl.BlockSpec((1, 2000, _H4), lambda bi, i: (bi, i, 0))],
        out_specs=pl.BlockSpec((1, 2000, 4), lambda bi, i: (bi, i, 0)),
        out_shape=jax.ShapeDtypeStruct((_B, _N, 4), _f32),
    )(gc)


_mesh = plsc.VectorSubcoreMesh(
    core_axis_name="c", subcore_axis_name="s", num_cores=_NC, num_subcores=_NS
)

_SC_SCRATCH = [
    pltpu.VMEM((_CAP, _CH), jnp.int32),
    pltpu.VMEM((_CAP, _CH), jnp.int32),
    pltpu.VMEM((16,), jnp.int32),
    pltpu.VMEM((_CH, 128), _f32),
    pltpu.VMEM((_CH, 128), _f32),
    pltpu.VMEM_SHARED((_RW + 8, 128), _f32),
    pltpu.SemaphoreType.DMA,
    pltpu.SemaphoreType.DMA,
]


def _range_pass(p, ysl, rsl, osl, srcm, dstm, ncnt, src_v, dst_v, cnt_v,
                rows0, rows1, agg, sem0, sem1, s):
    base = pl.multiple_of(p * _RW + s * _STR, 8)
    pltpu.sync_copy(srcm.at[p].at[s], src_v)
    pltpu.sync_copy(dstm.at[p].at[s], dst_v)
    pltpu.sync_copy(ncnt.at[p].at[s], cnt_v)
    pltpu.sync_copy(rsl.at[pl.ds(base, _STR)], agg.at[pl.ds(s * _STR, _STR)])
    n = jnp.max(cnt_v[...])
    plsc.subcore_barrier()
    pltpu.async_copy(ysl.at[src_v.at[0]], rows0, sem0)
    pltpu.async_copy(ysl.at[src_v.at[1]], rows1, sem1)

    def chunk_pair(i, carry):
        j0 = 2 * i
        pltpu.make_async_copy(ysl.at[src_v.at[j0]], rows0, sem0).wait()
        pltpu.sync_copy(rows0, agg.at[dst_v.at[j0]], add=True)
        pltpu.async_copy(ysl.at[src_v.at[lax.rem(j0 + 2, n)]], rows0, sem0)
        j1 = j0 + 1
        pltpu.make_async_copy(ysl.at[src_v.at[j1]], rows1, sem1).wait()
        pltpu.sync_copy(rows1, agg.at[dst_v.at[j1]], add=True)
        pltpu.async_copy(ysl.at[src_v.at[lax.rem(j1 + 2, n)]], rows1, sem1)
        return carry

    lax.fori_loop(0, n // 2, chunk_pair, 0)
    pltpu.make_async_copy(ysl.at[src_v.at[0]], rows0, sem0).wait()
    pltpu.make_async_copy(ysl.at[src_v.at[1]], rows1, sem1).wait()
    plsc.subcore_barrier()
    pltpu.sync_copy(agg.at[pl.ds(s * _STR, _STR)], osl.at[pl.ds(base, _STR)])
    plsc.subcore_barrier()


@functools.partial(
    pl.kernel,
    out_type=jax.ShapeDtypeStruct((_B, _NC, _NP, 128), _f32),
    mesh=_mesh,
    scratch_types=_SC_SCRATCH,
    compiler_params=pltpu.CompilerParams(needs_layout_passes=False),
)
def _sc_mid(y_hbm, r_hbm, srcm, dstm, ncnt, out_hbm, src_v, dst_v, cnt_v,
            rows0, rows1, agg, sem0, sem1):
    c = lax.axis_index("c")
    s = lax.axis_index("s")
    for b in range(_B):
        for u in range(4):
            h = u // 2
            if h == 0:
                p = c + 2 * (u % 2)
            else:
                p = (1 - c) + 2 * (u % 2)
            _range_pass(p, y_hbm.at[b].at[h], r_hbm.at[b].at[h],
                        out_hbm.at[b].at[h], srcm, dstm, ncnt, src_v, dst_v,
                        cnt_v, rows0, rows1, agg, sem0, sem1, s)


@functools.partial(
    pl.kernel,
    out_type=jax.ShapeDtypeStruct((_NP, 128), _f32),
    mesh=_mesh,
    scratch_types=_SC_SCRATCH,
    compiler_params=pltpu.CompilerParams(needs_layout_passes=False),
)
def _sc4(y_hbm, r_hbm, srcm, dstm, ncnt, out_hbm, src_v, dst_v, cnt_v,
         rows0, rows1, agg, sem0, sem1):
    c = lax.axis_index("c")
    s = lax.axis_index("s")
    for k in range(_NR // _NC):
        p = c + _NC * k
        _range_pass(p, y_hbm, r_hbm, out_hbm, srcm, dstm, ncnt, src_v, dst_v,
                    cnt_v, rows0, rows1, agg, sem0, sem1, s)


def _prep_edges(edge):
    src = edge[:, 0]
    dst = edge[:, 1]
    q = dst // _RW
    dstl = dst - q * _RW
    onehot = q[None, :] == jnp.arange(_NR, dtype=jnp.int32)[:, None]
    ranks = jnp.cumsum(onehot.astype(jnp.int32), axis=1) - 1
    rank = ranks[q, jnp.arange(_E)]
    g = rank // _CH
    tile = g % _NS
    slot = g // _NS
    flat = ((q * _NS + tile) * _CAP + slot) * _CH + rank % _CH
    size = _NR * _NS * _CAP * _CH
    pad_src = (jnp.arange(size, dtype=jnp.int32) % 640)
    srcm = (pad_src.at[flat].set(src, unique_indices=True,
                                 mode="promise_in_bounds")
            .reshape(_NR, _NS, _CAP, _CH))
    dstm = (jnp.full((size,), _TRASH, jnp.int32)
            .at[flat].set(dstl, unique_indices=True,
                          mode="promise_in_bounds")
            .reshape(_NR, _NS, _CAP, _CH))
    counts = jnp.sum(onehot, axis=1)
    nch = (counts + _CH - 1) // _CH
    svec = jnp.arange(_NS, dtype=jnp.int32)
    cnt = jnp.clip((nch[:, None] - svec[None, :] + _NS - 1) // _NS, 0)
    cnt = jnp.maximum(cnt + cnt % 2, 2)
    ncnt = jnp.broadcast_to(cnt[:, :, None], (_NR, _NS, 16)).astype(jnp.int32)
    return srcm, dstm, ncnt


def kernel(points, cs, edge, W1r, W1n, b1, W2r, W2n, b2, W3r, W3n, b3,
           W4r, W4n, b4):
    srcm, dstm, ncnt = _prep_edges(edge)
    pts_p = jnp.pad(points, ((0, 0), (0, _NP - _N), (0, 0)))
    css = jnp.pad(cs.reshape(_B, _N, _NC, 128).transpose(0, 2, 1, 3),
                  ((0, 0), (0, 0), (0, _NP - _N), (0, 0)))

    y1, r1 = _tc1(pts_p, W1r, W1n, b1.reshape(1, -1))
    gc1 = _sc_mid(y1, r1, srcm, dstm, ncnt)
    x1, y2, r2 = _tc_mid(gc1, css, W2r, W2n, b2.reshape(1, -1))
    gc2 = _sc_mid(y2, r2, srcm, dstm, ncnt)
    x2, y3, r3 = _tc_mid(gc2, x1, W3r, W3n, b3.reshape(1, -1))
    gc3 = _sc_mid(y3, r3, srcm, dstm, ncnt)

    W4r_p = jnp.zeros((_H, _H4), _f32).at[:, :4].set(W4r)
    W4n_p = jnp.zeros((_H, _H4), _f32).at[:, :4].set(W4n)
    b4_p = jnp.zeros((1, _H4), _f32).at[0, :4].set(b4)
    y4, r4 = _tc4(gc3, x2, W4r_p, W4n_p, b4_p)
    y4p = y4.transpose(1, 0, 2).reshape(_NP, _B * _H4)
    r4p = r4.transpose(1, 0, 2).reshape(_NP, _B * _H4)
    gc4p = _sc4(y4p, r4p, srcm, dstm, ncnt)
    gc4 = gc4p.reshape(_NP, _B, _H4).transpose(1, 0, 2)
    return _tc5(gc4)
